# baseline (speedup 1.0000x reference)
import functools
import jax
import jax.numpy as jnp
from jax import lax
from jax.experimental import pallas as pl
from jax.experimental.pallas import tpu as pltpu
from jax.experimental.pallas import tpu_sc as plsc

_B, _LEN, _D, _V = 4096, 200, 64, 1000000
_NC, _NS = 2, 16
_NW = _NC * _NS
_BPW = _B // _NW
_RPW = _BPW * _LEN
_GU = 100
_NBC = 2
_CH = _NBC * _LEN
_UPC = _CH // _GU
_NCHUNK = _BPW // _NBC


def _sig_body(y_ref, w_ref, b_ref, o_ref):
    o_ref[...] = (
        jnp.dot(y_ref[...], w_ref[...].T, preferred_element_type=jnp.float32)
        + b_ref[...]
    )


def _compute_sig(y, w, bias):
    return pl.pallas_call(
        _sig_body,
        out_shape=jax.ShapeDtypeStruct((_B, _D), jnp.float32),
    )(y, w, bias.reshape(1, _D))


@functools.partial(
    pl.kernel,
    out_type=jax.ShapeDtypeStruct((_B * _LEN, _D), jnp.float32),
    mesh=plsc.VectorSubcoreMesh(
        core_axis_name="c", subcore_axis_name="s", num_cores=_NC, num_subcores=_NS
    ),
    scratch_types=[
        pltpu.VMEM((_RPW // _GU, _GU), jnp.int32),
        pltpu.VMEM((_BPW, _D), jnp.float32),
        pltpu.VMEM((_CH, _D), jnp.float32),
        pltpu.SemaphoreType.DMA,
    ],
    compiler_params=pltpu.CompilerParams(use_tc_tiling_on_sc=False),
)
def _sc_embed(idx_hbm, sig_hbm, table_hbm, out_hbm, idx_v, sig_v, rows_v, gsem):
    wid = lax.axis_index("s") * _NC + lax.axis_index("c")
    base_row = wid * _RPW

    pltpu.sync_copy(idx_hbm.at[wid], idx_v)
    pltpu.sync_copy(sig_hbm.at[wid], sig_v)

    def chunk_body(c, carry):
        handles = []
        for u in range(_UPC):
            handles.append(
                pltpu.async_copy(
                    table_hbm.at[idx_v.at[c * _UPC + u]],
                    rows_v.at[pl.ds(u * _GU, _GU)],
                    gsem,
                )
            )
        for h in handles:
            h.wait()

        for ib in range(_NBC):
            b = c * _NBC + ib
            svs = [sig_v[b, pl.ds(d * 16, 16)] for d in range(_D // 16)]

            def row_body(r, carry2, _ib=ib, _svs=svs):
                for d in range(_D // 16):
                    plsc.addupdate(
                        rows_v.at[_ib * _LEN + r, pl.ds(d * 16, 16)], _svs[d]
                    )
                return carry2

            lax.fori_loop(0, _LEN, row_body, 0, unroll=4)

        pltpu.sync_copy(rows_v, out_hbm.at[pl.ds(base_row + c * _CH, _CH)])
        return carry

    lax.fori_loop(0, _NCHUNK, chunk_body, 0)


@jax.jit
def kernel(x, y, embedding, W, b):
    sig = _compute_sig(y, W, b)
    idx = x.reshape(_NW, _RPW // _GU, _GU)
    sigw = sig.reshape(_NW, _BPW, _D)
    out = _sc_embed(idx, sigw, embedding)
    return out.reshape(_B, _LEN, _D)


# speedup vs baseline: 1.0803x; 1.0803x over previous
import functools
import jax
import jax.numpy as jnp
from jax import lax
from jax.experimental import pallas as pl
from jax.experimental.pallas import tpu as pltpu
from jax.experimental.pallas import tpu_sc as plsc

_B, _LEN, _D, _V = 4096, 200, 64, 1000000
_NC, _NS = 2, 16
_NW = _NC * _NS
_BPW = _B // _NW
_RPW = _BPW * _LEN
_GU = 100
_NBC = 2
_CH = _NBC * _LEN
_UPC = _CH // _GU
_NCHUNK = _BPW // _NBC


def _sig_body(y_ref, w_ref, b_ref, o_ref):
    o_ref[...] = (
        jnp.dot(y_ref[...], w_ref[...].T, preferred_element_type=jnp.float32)
        + b_ref[...]
    )


def _compute_sig(y, w, bias):
    return pl.pallas_call(
        _sig_body,
        out_shape=jax.ShapeDtypeStruct((_B, _D), jnp.float32),
    )(y, w, bias.reshape(1, _D))


@functools.partial(
    pl.kernel,
    out_type=jax.ShapeDtypeStruct((_B * _LEN, _D), jnp.float32),
    mesh=plsc.VectorSubcoreMesh(
        core_axis_name="c", subcore_axis_name="s", num_cores=_NC, num_subcores=_NS
    ),
    scratch_types=[
        pltpu.VMEM((_RPW // _GU, _GU), jnp.int32),
        pltpu.VMEM((_BPW, _D), jnp.float32),
        pltpu.VMEM((_CH, _D), jnp.float32),
        pltpu.VMEM((_CH, _D), jnp.float32),
        pltpu.SemaphoreType.DMA,
        pltpu.SemaphoreType.DMA,
        pltpu.SemaphoreType.DMA,
        pltpu.SemaphoreType.DMA,
    ],
    compiler_params=pltpu.CompilerParams(use_tc_tiling_on_sc=False),
)
def _sc_embed(
    idx_hbm, sig_hbm, table_hbm, out_hbm,
    idx_v, sig_v, buf0, buf1, gsem0, gsem1, osem0, osem1,
):
    wid = lax.axis_index("s") * _NC + lax.axis_index("c")
    base_row = wid * _RPW

    pltpu.sync_copy(idx_hbm.at[wid], idx_v)
    pltpu.sync_copy(sig_hbm.at[wid], sig_v)

    def fire_gather(c, buf, gsem):
        for u in range(_UPC):
            pltpu.async_copy(
                table_hbm.at[idx_v.at[c * _UPC + u]],
                buf.at[pl.ds(u * _GU, _GU)],
                gsem,
            )

    def wait_gather(c, buf, gsem):
        for u in range(_UPC):
            pltpu.make_async_copy(
                table_hbm.at[idx_v.at[c * _UPC + u]],
                buf.at[pl.ds(u * _GU, _GU)],
                gsem,
            ).wait()

    def fire_write(c, buf, osem):
        pltpu.async_copy(buf, out_hbm.at[pl.ds(base_row + c * _CH, _CH)], osem)

    def wait_write(buf, osem):
        pltpu.make_async_copy(buf, out_hbm.at[pl.ds(base_row, _CH)], osem).wait()

    def add_sig(c, buf):
        for ib in range(_NBC):
            b = c * _NBC + ib
            svs = [sig_v[b, pl.ds(d * 16, 16)] for d in range(_D // 16)]

            def row_body(r, carry2, _ib=ib, _svs=svs):
                for d in range(_D // 16):
                    plsc.addupdate(
                        buf.at[_ib * _LEN + r, pl.ds(d * 16, 16)], _svs[d]
                    )
                return carry2

            lax.fori_loop(0, _LEN, row_body, 0, unroll=8)

    fire_gather(0, buf0, gsem0)

    def pair_body(i, carry):
        c0 = 2 * i
        c1 = c0 + 1

        @pl.when(i > 0)
        def _():
            wait_write(buf1, osem1)
        fire_gather(c1, buf1, gsem1)
        wait_gather(c0, buf0, gsem0)
        add_sig(c0, buf0)
        fire_write(c0, buf0, osem0)

        @pl.when(i < _NCHUNK // 2 - 1)
        def _():
            wait_write(buf0, osem0)
            fire_gather(c0 + 2, buf0, gsem0)
        wait_gather(c1, buf1, gsem1)
        add_sig(c1, buf1)
        fire_write(c1, buf1, osem1)
        return carry

    lax.fori_loop(0, _NCHUNK // 2, pair_body, 0)

    wait_write(buf0, osem0)
    wait_write(buf1, osem1)


@jax.jit
def kernel(x, y, embedding, W, b):
    sig = _compute_sig(y, W, b)
    idx = x.reshape(_NW, _RPW // _GU, _GU)
    sigw = sig.reshape(_NW, _BPW, _D)
    out = _sc_embed(idx, sigw, embedding)
    return out.reshape(_B, _LEN, _D)
